# Initial kernel scaffold; baseline (speedup 1.0000x reference)
import jax
import jax.numpy as jnp
from jax.experimental import pallas as pl


def kernel(x0, edge_index0, edge_attr0, batch0, x1, edge_index1, edge_attr1, batch1, params):
    raise NotImplementedError("write your pallas kernel here")


# speedup vs baseline: 5.2955x; 5.2955x over previous
import functools

import jax
import jax.numpy as jnp
from jax import lax
from jax.experimental import pallas as pl
from jax.experimental.pallas import tpu as pltpu
from jax.experimental.pallas import tpu_sc as plsc

_HI = lax.Precision.HIGHEST

N = 10000
E = 160000
G = 128
D = 304
H = 152
DH = 608
BN = 1000
NB = N // BN
NSC = 16
EPT = E // NSC
CH = 80
NCHUNK = EPT // CH
NP = N
RPT = NP // NSC
SRO = 125
CSZ = 2000
C16 = 16


_MESH = plsc.VectorSubcoreMesh(core_axis_name="c", subcore_axis_name="s")
_SC_PARAMS = pltpu.CompilerParams(use_tc_tiling_on_sc=False)


def _spmm_body(hlo, hhi, src, dst, zer, out, idx_s, idx_d, rows, stg, acc,
               sem):
    cid = lax.axis_index("c")
    sid = lax.axis_index("s")
    r0 = sid * RPT
    pltpu.sync_copy(zer, stg)

    def zinit(i, carry):
        pltpu.sync_copy(stg, acc.at[pl.ds(r0 + i * SRO, SRO)])
        return carry

    lax.fori_loop(0, RPT // SRO, zinit, 0)
    plsc.subcore_barrier()
    ebase = sid * EPT

    def chunk(i, carry):
        b = ebase + i * CH
        pltpu.sync_copy(src.at[pl.ds(b, CH)], idx_s)
        pltpu.sync_copy(dst.at[pl.ds(b, CH)], idx_d)

        @pl.when(cid == 0)
        def _():
            pltpu.async_copy(hlo.at[idx_s], rows, sem).wait()

        @pl.when(cid == 1)
        def _():
            pltpu.async_copy(hhi.at[idx_s], rows, sem).wait()

        pltpu.sync_copy(rows, acc.at[idx_d], add=True)
        return carry

    lax.fori_loop(0, NCHUNK, chunk, 0)
    plsc.subcore_barrier()

    def wout(i, carry):
        pltpu.sync_copy(acc.at[pl.ds(r0 + i * SRO, SRO)], stg)
        pltpu.sync_copy(stg, out.at[cid, pl.ds(r0 + i * SRO, SRO)])
        return carry

    lax.fori_loop(0, RPT // SRO, wout, 0)


_spmm = functools.partial(
    pl.kernel,
    mesh=_MESH,
    out_type=jax.ShapeDtypeStruct((2, NP, H), jnp.float32),
    scratch_types=[
        pltpu.VMEM((CH,), jnp.int32),
        pltpu.VMEM((CH,), jnp.int32),
        pltpu.VMEM((CH, H), jnp.float32),
        pltpu.VMEM((SRO, H), jnp.float32),
        pltpu.VMEM_SHARED((NP, H), jnp.float32),
        pltpu.SemaphoreType.DMA,
    ],
    compiler_params=_SC_PARAMS,
)(_spmm_body)


def _counts_body(bt, bd, dst, zer, out, btv, bdv, dstv, idx1, idx2, ones,
                 cstg, acc):
    cid = lax.axis_index("c")
    sid = lax.axis_index("s")
    r0 = sid * (N * C16 // NSC)
    pltpu.sync_copy(zer.at[pl.ds(0, CSZ)], cstg)

    def zinit(i, carry):
        pltpu.sync_copy(cstg, acc.at[pl.ds(r0 + i * CSZ, CSZ)])
        return carry

    lax.fori_loop(0, (N * C16 // NSC) // CSZ, zinit, 0)
    for j in range(CH // 16):
        ones[pl.ds(16 * j, 16)] = jnp.ones((16,), jnp.float32)
    plsc.subcore_barrier()
    wid = cid * NSC + sid
    nfull = (E // CH) // 32
    nextra = (E // CH) - nfull * 32
    nchunks = nfull + jnp.where(wid < nextra, 1, 0)

    def chunk(i, carry):
        b = (wid + i * 32) * CH
        pltpu.sync_copy(bt.at[pl.ds(b, CH)], btv)
        pltpu.sync_copy(bd.at[pl.ds(b, CH)], bdv)
        pltpu.sync_copy(dst.at[pl.ds(b, CH)], dstv)
        for j in range(CH // 16):
            s = pl.ds(16 * j, 16)
            base16 = dstv[s] * C16
            idx1[s] = base16 + btv[s]
            idx2[s] = base16 + 8 + bdv[s]
        pltpu.sync_copy(ones, acc.at[idx1], add=True)
        pltpu.sync_copy(ones, acc.at[idx2], add=True)
        return carry

    lax.fori_loop(0, nchunks, chunk, 0)
    plsc.subcore_barrier()

    def wout(i, carry):
        pltpu.sync_copy(acc.at[pl.ds(r0 + i * CSZ, CSZ)], cstg)
        pltpu.sync_copy(cstg, out.at[pl.ds(cid * (N * C16) + r0 + i * CSZ,
                                           CSZ)])
        return carry

    lax.fori_loop(0, (N * C16 // NSC) // CSZ, wout, 0)


_counts = functools.partial(
    pl.kernel,
    mesh=_MESH,
    out_type=jax.ShapeDtypeStruct((2 * N * C16,), jnp.float32),
    scratch_types=[
        pltpu.VMEM((CH,), jnp.int32),
        pltpu.VMEM((CH,), jnp.int32),
        pltpu.VMEM((CH,), jnp.int32),
        pltpu.VMEM((CH,), jnp.int32),
        pltpu.VMEM((CH,), jnp.int32),
        pltpu.VMEM((CH,), jnp.float32),
        pltpu.VMEM((CSZ,), jnp.float32),
        pltpu.VMEM_SHARED((N * C16,), jnp.float32),
    ],
    compiler_params=_SC_PARAMS,
)(_counts_body)


def _emb_body(xa_ref, xb_ref, e1_ref, e2_ref, h2_ref):
    lanes = lax.broadcasted_iota(jnp.int32, (BN, 128), 1)
    oh1 = (xa_ref[...] == lanes).astype(jnp.float32)
    oh2 = (xb_ref[...] == lanes).astype(jnp.float32)
    h0 = jnp.dot(oh1, e1_ref[...], preferred_element_type=jnp.float32, precision=_HI)
    h0 = h0 + jnp.dot(oh2, e2_ref[...], preferred_element_type=jnp.float32, precision=_HI)
    h2_ref[0] = h0[:, :H]
    h2_ref[1] = h0[:, H:]


def _embed(xa, xb, e1p, e2p):
    return pl.pallas_call(
        _emb_body,
        grid=(NB,),
        in_specs=[
            pl.BlockSpec((BN, 1), lambda i: (i, 0)),
            pl.BlockSpec((BN, 1), lambda i: (i, 0)),
            pl.BlockSpec((128, D), lambda i: (0, 0)),
            pl.BlockSpec((128, D), lambda i: (0, 0)),
        ],
        out_specs=pl.BlockSpec((2, BN, H), lambda i: (0, i, 0)),
        out_shape=jax.ShapeDtypeStruct((2, NP, H), jnp.float32),
    )(xa, xb, e1p, e2p)


def _layerA_body(agg_ref, hp_ref, cnt_ref, bm_ref, slc_ref,
                 w1_ref, b1_ref, w2_ref, b2_ref,
                 hn_ref, ssum_ref, ssq_ref):
    i = pl.program_id(0)
    aggr = jnp.concatenate([agg_ref[0], agg_ref[1]], axis=1)
    aggr = aggr + jnp.concatenate([hp_ref[0], hp_ref[1]], axis=1)
    cnt = cnt_ref[0] + cnt_ref[1]
    aggr = aggr + jnp.dot(cnt, bm_ref[...], preferred_element_type=jnp.float32, precision=_HI)
    aggr = aggr + slc_ref[...]
    hm = jnp.dot(aggr, w1_ref[...], preferred_element_type=jnp.float32)
    hm = jnp.maximum(hm + b1_ref[...], 0.0)
    hn = jnp.dot(hm, w2_ref[...], preferred_element_type=jnp.float32)
    hn = hn + b2_ref[...]
    hn_ref[...] = hn

    @pl.when(i == 0)
    def _():
        ssum_ref[...] = jnp.zeros_like(ssum_ref)
        ssq_ref[...] = jnp.zeros_like(ssq_ref)

    ssum_ref[...] += jnp.sum(hn, axis=0, keepdims=True)
    ssq_ref[...] += jnp.sum(hn * hn, axis=0, keepdims=True)


def _layerA(agg2, hp2, cnt2, bm, slc, w1, b1, w2, b2):
    return pl.pallas_call(
        _layerA_body,
        grid=(NB,),
        in_specs=[
            pl.BlockSpec((2, BN, H), lambda i: (0, i, 0)),
            pl.BlockSpec((2, BN, H), lambda i: (0, i, 0)),
            pl.BlockSpec((2, BN, C16), lambda i: (0, i, 0)),
            pl.BlockSpec((C16, D), lambda i: (0, 0)),
            pl.BlockSpec((1, D), lambda i: (0, 0)),
            pl.BlockSpec((D, DH), lambda i: (0, 0)),
            pl.BlockSpec((1, DH), lambda i: (0, 0)),
            pl.BlockSpec((DH, D), lambda i: (0, 0)),
            pl.BlockSpec((1, D), lambda i: (0, 0)),
        ],
        out_specs=[
            pl.BlockSpec((BN, D), lambda i: (i, 0)),
            pl.BlockSpec((1, D), lambda i: (0, 0)),
            pl.BlockSpec((1, D), lambda i: (0, 0)),
        ],
        out_shape=[
            jax.ShapeDtypeStruct((N, D), jnp.float32),
            jax.ShapeDtypeStruct((1, D), jnp.float32),
            jax.ShapeDtypeStruct((1, D), jnp.float32),
        ],
    )(agg2, hp2, cnt2, bm, slc, w1, b1, w2, b2)


def _layerB_body(hn_ref, ssum_ref, ssq_ref, bnw_ref, bnb_ref, h2_ref, *,
                 relu):
    mean = ssum_ref[...] * jnp.float32(1.0 / N)
    var = ssq_ref[...] * jnp.float32(1.0 / N) - mean * mean
    y = ((hn_ref[...] - mean) / jnp.sqrt(var + 1e-5)) * bnw_ref[...] \
        + bnb_ref[...]
    if relu:
        y = jnp.maximum(y, 0.0)
    h2_ref[0] = y[:, :H]
    h2_ref[1] = y[:, H:]


def _layerB(hn, ssum, ssq, bnw, bnb, relu):
    return pl.pallas_call(
        functools.partial(_layerB_body, relu=relu),
        grid=(NB,),
        in_specs=[
            pl.BlockSpec((BN, D), lambda i: (i, 0)),
            pl.BlockSpec((1, D), lambda i: (0, 0)),
            pl.BlockSpec((1, D), lambda i: (0, 0)),
            pl.BlockSpec((1, D), lambda i: (0, 0)),
            pl.BlockSpec((1, D), lambda i: (0, 0)),
        ],
        out_specs=pl.BlockSpec((2, BN, H), lambda i: (0, i, 0)),
        out_shape=jax.ShapeDtypeStruct((2, NP, H), jnp.float32),
    )(hn, ssum, ssq, bnw, bnb)


def _pool_body(h0_ref, h1_ref, b0_ref, b1_ref, pw_ref, pb_ref, out_ref,
               s0, c0, s1, c1):
    i = pl.program_id(0)

    @pl.when(i == 0)
    def _():
        s0[...] = jnp.zeros_like(s0)
        c0[...] = jnp.zeros_like(c0)
        s1[...] = jnp.zeros_like(s1)
        c1[...] = jnp.zeros_like(c1)

    lanes = lax.broadcasted_iota(jnp.int32, (BN, G), 1)
    oh0 = (b0_ref[...] == lanes).astype(jnp.float32)
    oh1 = (b1_ref[...] == lanes).astype(jnp.float32)
    hb0 = jnp.concatenate([h0_ref[0], h0_ref[1]], axis=1)
    hb1 = jnp.concatenate([h1_ref[0], h1_ref[1]], axis=1)
    dn = (((0,), (0,)), ((), ()))
    s0[...] += lax.dot_general(oh0, hb0, dn, preferred_element_type=jnp.float32, precision=_HI)
    s1[...] += lax.dot_general(oh1, hb1, dn, preferred_element_type=jnp.float32, precision=_HI)
    c0[...] += jnp.sum(oh0, axis=0, keepdims=True)
    c1[...] += jnp.sum(oh1, axis=0, keepdims=True)

    @pl.when(i == NB - 1)
    def _():
        m0 = s0[...] / jnp.maximum(c0[...], 1.0).T
        m1 = s1[...] / jnp.maximum(c1[...], 1.0).T
        n0 = jnp.sqrt(jnp.sum(m0 * m0, axis=1, keepdims=True))
        f0 = m0 / jnp.maximum(n0, 1e-12)
        p1 = jnp.dot(m1, pw_ref[...], preferred_element_type=jnp.float32)
        p1 = p1 + pb_ref[...]
        n1 = jnp.sqrt(jnp.sum(p1 * p1, axis=1, keepdims=True))
        f1 = p1 / jnp.maximum(n1, 1e-12)
        dn2 = (((1,), (1,)), ((), ()))
        out_ref[...] = lax.dot_general(
            f0, f1, dn2, preferred_element_type=jnp.float32) / 0.04


def _pool(h0_2, h1_2, b0, b1, pw, pbias):
    return pl.pallas_call(
        _pool_body,
        grid=(NB,),
        in_specs=[
            pl.BlockSpec((2, BN, H), lambda i: (0, i, 0)),
            pl.BlockSpec((2, BN, H), lambda i: (0, i, 0)),
            pl.BlockSpec((BN, 1), lambda i: (i, 0)),
            pl.BlockSpec((BN, 1), lambda i: (i, 0)),
            pl.BlockSpec((D, D), lambda i: (0, 0)),
            pl.BlockSpec((1, D), lambda i: (0, 0)),
        ],
        out_specs=pl.BlockSpec((G, G), lambda i: (0, 0)),
        out_shape=jax.ShapeDtypeStruct((G, G), jnp.float32),
        scratch_shapes=[
            pltpu.VMEM((G, D), jnp.float32),
            pltpu.VMEM((1, G), jnp.float32),
            pltpu.VMEM((G, D), jnp.float32),
            pltpu.VMEM((1, G), jnp.float32),
        ],
    )(h0_2, h1_2, b0, b1, pw, pbias)


def _pad2(a, r, c):
    return jnp.zeros((r, c), jnp.float32).at[: a.shape[0], : a.shape[1]].set(a)


def _pad_row(a, c):
    return jnp.zeros((1, c), jnp.float32).at[0, : a.shape[0]].set(a)


def _branch(x, edge_index, edge_attr, params, zeros_h, zeros_c):
    src = edge_index[0].astype(jnp.int32)
    dst = edge_index[1].astype(jnp.int32)
    bt = edge_attr[:, 0].astype(jnp.int32)
    bd = edge_attr[:, 1].astype(jnp.int32)
    xa = x[:, 0].astype(jnp.int32).reshape(N, 1)
    xb = x[:, 1].astype(jnp.int32).reshape(N, 1)

    e1p = _pad2(params["atom_emb1"], 128, D)
    e2p = _pad2(params["atom_emb2"], 128, D)
    h2 = _embed(xa, xb, e1p, e2p)

    cnt = _counts(bt, bd, dst, zeros_c)
    cnt2 = cnt.reshape(2, N, C16)

    for l in range(5):
        lp = params["layers"][l]
        bm = jnp.concatenate([
            _pad2(lp["bond_emb1"], 8, D),
            _pad2(lp["bond_emb2"], 8, D),
        ], axis=0)
        slc = _pad_row(lp["bond_emb1"][4] + lp["bond_emb2"][0], D)
        w1 = _pad2(lp["W1"], D, DH)
        b1 = _pad_row(lp["b1"], DH)
        w2 = _pad2(lp["W2"], DH, D)
        b2 = _pad_row(lp["b2"], D)
        bnw = _pad_row(lp["bn_w"], D)
        bnb = _pad_row(lp["bn_b"], D)

        agg2 = _spmm(h2[0], h2[1], src, dst, zeros_h)
        hn, ssum, ssq = _layerA(agg2, h2, cnt2, bm, slc, w1, b1, w2, b2)
        h2 = _layerB(hn, ssum, ssq, bnw, bnb, relu=(l != 4))
    return h2


def kernel(x0, edge_index0, edge_attr0, batch0,
           x1, edge_index1, edge_attr1, batch1, params):
    zeros_h = jnp.zeros((SRO, H), jnp.float32)
    zeros_c = jnp.zeros((N * C16,), jnp.float32)
    h0_2 = _branch(x0, edge_index0, edge_attr0, params, zeros_h, zeros_c)
    h1_2 = _branch(x1, edge_index1, edge_attr1, params, zeros_h, zeros_c)
    b0 = batch0.astype(jnp.int32).reshape(N, 1)
    b1 = batch1.astype(jnp.int32).reshape(N, 1)
    pw = _pad2(params["proj_W"], D, D)
    pb = _pad_row(params["proj_b"], D)
    return _pool(h0_2, h1_2, b0, b1, pw, pb)
